# Initial kernel scaffold; baseline (speedup 1.0000x reference)
import jax
import jax.numpy as jnp
from jax.experimental import pallas as pl


def kernel(inputs, embedding):
    raise NotImplementedError("write your pallas kernel here")


# speedup vs baseline: 1.4698x; 1.4698x over previous
import functools

import jax
import jax.numpy as jnp
from jax import lax
from jax.experimental import pallas as pl
from jax.experimental.pallas import tpu as pltpu
from jax.experimental.pallas import tpu_sc as plsc

NUM_EMBEDDINGS = 1000000
FEATURES = 32
BATCH = 4096
LENGTH = 200

NC = 2
NS = 16
NW = NC * NS

B = BATCH * LENGTH
B_PER_W = B // NW
CHUNK = 1280
NCHUNK = B_PER_W // CHUNK


def _make_gather():
    mesh = plsc.VectorSubcoreMesh(
        core_axis_name="c", subcore_axis_name="s", num_cores=NC, num_subcores=NS
    )

    @functools.partial(
        pl.kernel,
        out_type=jax.ShapeDtypeStruct((B, FEATURES), jnp.float32),
        mesh=mesh,
        compiler_params=pltpu.CompilerParams(use_tc_tiling_on_sc=False),
        scratch_types=[
            pltpu.VMEM((CHUNK,), jnp.int32),
            pltpu.VMEM((CHUNK, FEATURES), jnp.float32),
            pltpu.SemaphoreType.DMA,
        ],
    )
    def k(table_hbm, idx_hbm, out_hbm, idx_v, rows_v, gsem):
        wid = lax.axis_index("s") * NC + lax.axis_index("c")

        def chunk_body(ci, carry):
            pltpu.sync_copy(idx_hbm.at[wid, ci], idx_v)
            pltpu.async_copy(table_hbm.at[idx_v], rows_v, gsem).wait()
            pltpu.sync_copy(
                rows_v, out_hbm.at[pl.ds((wid * NCHUNK + ci) * CHUNK, CHUNK)]
            )
            return carry

        lax.fori_loop(0, NCHUNK, chunk_body, 0)

    return k


_gather = _make_gather()


def kernel(inputs, embedding):
    idx = inputs.reshape(NW, NCHUNK, CHUNK)
    out = _gather(embedding, idx)
    return out.reshape(BATCH, LENGTH, FEATURES)


# speedup vs baseline: 1.4984x; 1.0194x over previous
import functools

import jax
import jax.numpy as jnp
from jax import lax
from jax.experimental import pallas as pl
from jax.experimental.pallas import tpu as pltpu
from jax.experimental.pallas import tpu_sc as plsc

NUM_EMBEDDINGS = 1000000
FEATURES = 32
BATCH = 4096
LENGTH = 200

NC = 2
NS = 16
NW = NC * NS

B = BATCH * LENGTH
B_PER_W = B // NW
CHUNK = 1280
NCHUNK = B_PER_W // CHUNK
NBUF = 2


def _make_gather():
    mesh = plsc.VectorSubcoreMesh(
        core_axis_name="c", subcore_axis_name="s", num_cores=NC, num_subcores=NS
    )

    idx_scratch = [pltpu.VMEM((CHUNK,), jnp.int32) for _ in range(NCHUNK)]
    row_scratch = [pltpu.VMEM((CHUNK, FEATURES), jnp.float32) for _ in range(NBUF)]

    @functools.partial(
        pl.kernel,
        out_type=jax.ShapeDtypeStruct((B, FEATURES), jnp.float32),
        mesh=mesh,
        compiler_params=pltpu.CompilerParams(use_tc_tiling_on_sc=False),
        scratch_types=idx_scratch
        + row_scratch
        + [pltpu.SemaphoreType.DMA] * (1 + 2 * NBUF),
    )
    def k(table_hbm, idx_hbm, out_hbm, *refs):
        idx_v = refs[:NCHUNK]
        rows_v = refs[NCHUNK : NCHUNK + NBUF]
        sems = refs[NCHUNK + NBUF :]
        isem = sems[0]
        gsem = sems[1 : 1 + NBUF]
        wsem = sems[1 + NBUF :]
        wid = lax.axis_index("s") * NC + lax.axis_index("c")

        idx_descs = [
            pltpu.async_copy(idx_hbm.at[wid, c], idx_v[c], isem)
            for c in range(NCHUNK)
        ]
        for d in idx_descs:
            d.wait()

        def gather(c, b):
            return pltpu.async_copy(
                table_hbm.at[idx_v[c]], rows_v[b], gsem[b]
            )

        def write(c, b):
            return pltpu.async_copy(
                rows_v[b],
                out_hbm.at[pl.ds((wid * NCHUNK + c) * CHUNK, CHUNK)],
                wsem[b],
            )

        gd = [None] * NBUF
        wd = [None] * NBUF
        gd[0] = gather(0, 0)
        for c in range(NCHUNK):
            b = c % NBUF
            nb = (c + 1) % NBUF
            if c + 1 < NCHUNK:
                if wd[nb] is not None:
                    wd[nb].wait()
                gd[nb] = gather(c + 1, nb)
            gd[b].wait()
            wd[b] = write(c, b)
        for d in wd:
            if d is not None:
                d.wait()

    return k


_gather = _make_gather()


def kernel(inputs, embedding):
    idx = inputs.reshape(NW, NCHUNK, CHUNK)
    out = _gather(embedding, idx)
    return out.reshape(BATCH, LENGTH, FEATURES)
